# Initial kernel scaffold; baseline (speedup 1.0000x reference)
import functools

import jax
import jax.numpy as jnp
from jax import lax
from jax.experimental import pallas as pl
from jax.experimental.pallas import tpu as pltpu
from jax.experimental.pallas import tpu_sc as plsc

N_NODES = 10000
N_EDGES = 320000
D = 128
ROW = 2 * D

NC, NS, NL = 2, 16, 16
NW = NC * NS
EPW = N_EDGES // NW
K = 80
STEPS = EPW // K


def _tab_body(hs_ref, hd_ref, rel_ref, w1_ref, b1_ref, stab_ref, dtab_ref):
    w1a = w1_ref[0:D, :]
    w1b = w1_ref[D:2 * D, :]
    w1c = w1_ref[2 * D:3 * D, :]
    rel = rel_ref[:]
    c = jnp.dot(rel, w1c, preferred_element_type=jnp.float32) + b1_ref[:]
    hs = hs_ref[:]
    hd = hd_ref[:]
    stab_ref[:, 0:D] = jnp.dot(hs, w1a, preferred_element_type=jnp.float32) + c
    stab_ref[:, D:ROW] = hs * rel
    dtab_ref[:, 0:D] = jnp.dot(hd, w1b, preferred_element_type=jnp.float32)
    dtab_ref[:, D:ROW] = hd


def _build_tables(h_src, h_dst, rel2d, W1, b1_2d):
    blk = 1000
    grid = (N_NODES // blk,)
    return pl.pallas_call(
        _tab_body,
        grid=grid,
        in_specs=[
            pl.BlockSpec((blk, D), lambda i: (i, 0)),
            pl.BlockSpec((blk, D), lambda i: (i, 0)),
            pl.BlockSpec((1, D), lambda i: (0, 0)),
            pl.BlockSpec((3 * D, D), lambda i: (0, 0)),
            pl.BlockSpec((1, D), lambda i: (0, 0)),
        ],
        out_specs=[
            pl.BlockSpec((blk, ROW), lambda i: (i, 0)),
            pl.BlockSpec((blk, ROW), lambda i: (i, 0)),
        ],
        out_shape=[
            jax.ShapeDtypeStruct((N_NODES, ROW), jnp.float32),
            jax.ShapeDtypeStruct((N_NODES, ROW), jnp.float32),
        ],
    )(h_src, h_dst, rel2d, W1, b1_2d)


def _edge_body(src_idx_hbm, dst_idx_hbm, stab_hbm, dtab_hbm, w2_hbm, b2_hbm,
               out_hbm, sidx, didx, srows, drows, w2v, b2v, gbuf, pbuf, obuf,
               sem):
    wid = lax.axis_index("s") * NC + lax.axis_index("c")
    base = wid * EPW

    pltpu.sync_copy(w2_hbm, w2v)
    pltpu.sync_copy(b2_hbm, b2v)
    w2r = [w2v[pl.ds(NL * k, NL)] for k in range(D // NL)]
    b2r = b2v[...]
    iota = lax.iota(jnp.int32, NL)
    zero = jnp.zeros((NL,), jnp.float32)

    def step(s, _):
        eb = base + s * K
        pltpu.sync_copy(src_idx_hbm.at[pl.ds(eb, K)], sidx)
        pltpu.sync_copy(dst_idx_hbm.at[pl.ds(eb, K)], didx)
        pltpu.async_copy(stab_hbm.at[sidx], srows, sem).wait()
        pltpu.async_copy(dtab_hbm.at[didx], drows, sem).wait()

        def group(g, _):
            def edge(j, _):
                jj = g * NL + j
                acc_g = zero
                acc_p = zero
                for k in range(D // NL):
                    a = srows[jj, pl.ds(NL * k, NL)]
                    b = drows[jj, pl.ds(NL * k, NL)]
                    sv = srows[jj, pl.ds(D + NL * k, NL)]
                    hd = drows[jj, pl.ds(D + NL * k, NL)]
                    h = a + b
                    lh = jnp.maximum(h, 0.2 * h)
                    acc_g = acc_g + lh * w2r[k]
                    acc_p = acc_p + sv * hd
                gbuf[pl.ds(j * NL, NL)] = acc_g
                pbuf[pl.ds(j * NL, NL)] = acc_p
                return 0

            lax.fori_loop(0, NL, edge, 0)

            def red(d, carry):
                ga, pa = carry
                col = iota * NL + d
                ga = ga + plsc.load_gather(gbuf, [col])
                pa = pa + plsc.load_gather(pbuf, [col])
                return ga, pa

            gacc, pacc = lax.fori_loop(0, NL, red, (b2r, zero))
            gate = 1.0 / (1.0 + jnp.exp(-gacc))
            obuf[pl.ds(g * NL, NL)] = gate * pacc
            return 0

        lax.fori_loop(0, K // NL, group, 0)
        pltpu.sync_copy(obuf, out_hbm.at[pl.ds(eb, K)])
        return 0

    lax.fori_loop(0, STEPS, step, 0)


def _edge_kernel(src_idx, dst_idx, stab, dtab, w2, b2vec):
    mesh = plsc.VectorSubcoreMesh(core_axis_name="c", subcore_axis_name="s")
    return pl.kernel(
        _edge_body,
        out_type=jax.ShapeDtypeStruct((N_EDGES,), jnp.float32),
        mesh=mesh,
        scratch_types=[
            pltpu.VMEM((K,), jnp.int32),
            pltpu.VMEM((K,), jnp.int32),
            pltpu.VMEM((K, ROW), jnp.float32),
            pltpu.VMEM((K, ROW), jnp.float32),
            pltpu.VMEM((D,), jnp.float32),
            pltpu.VMEM((NL,), jnp.float32),
            pltpu.VMEM((NL * NL,), jnp.float32),
            pltpu.VMEM((NL * NL,), jnp.float32),
            pltpu.VMEM((K,), jnp.float32),
            pltpu.SemaphoreType.DMA,
        ],
    )(src_idx, dst_idx, stab, dtab, w2, b2vec)


def kernel(edge_index, h_src, h_dst, rel_weight, W1, b1, W2, b2):
    src_idx = edge_index[0].astype(jnp.int32)
    dst_idx = edge_index[1].astype(jnp.int32)
    rel2d = rel_weight.reshape(1, D)
    b1_2d = b1.reshape(1, D)
    stab, dtab = _build_tables(h_src, h_dst, rel2d, W1, b1_2d)
    w2 = W2.reshape(D)
    b2vec = jnp.broadcast_to(b2.reshape(()), (NL,))
    return _edge_kernel(src_idx, dst_idx, stab, dtab, w2, b2vec)


# speedup vs baseline: 2.7558x; 2.7558x over previous
import functools

import jax
import jax.numpy as jnp
from jax import lax
from jax.experimental import pallas as pl
from jax.experimental.pallas import tpu as pltpu
from jax.experimental.pallas import tpu_sc as plsc

N_NODES = 10000
N_EDGES = 320000
D = 128
ROW = 2 * D

NC, NS, NL = 2, 16, 16
NW = NC * NS
EPW = N_EDGES // NW
K = 80
STEPS = EPW // K


def _tab_body(hs_ref, hd_ref, rel_ref, w1_ref, b1_ref, stab_ref, dtab_ref):
    w1a = w1_ref[0:D, :]
    w1b = w1_ref[D:2 * D, :]
    w1c = w1_ref[2 * D:3 * D, :]
    rel = rel_ref[:]
    c = jnp.dot(rel, w1c, preferred_element_type=jnp.float32) + b1_ref[:]
    hs = hs_ref[:]
    hd = hd_ref[:]
    stab_ref[:, 0:D] = jnp.dot(hs, w1a, preferred_element_type=jnp.float32) + c
    stab_ref[:, D:ROW] = hs * rel
    dtab_ref[:, 0:D] = jnp.dot(hd, w1b, preferred_element_type=jnp.float32)
    dtab_ref[:, D:ROW] = hd


def _build_tables(h_src, h_dst, rel2d, W1, b1_2d):
    blk = 1000
    grid = (N_NODES // blk,)
    return pl.pallas_call(
        _tab_body,
        grid=grid,
        in_specs=[
            pl.BlockSpec((blk, D), lambda i: (i, 0)),
            pl.BlockSpec((blk, D), lambda i: (i, 0)),
            pl.BlockSpec((1, D), lambda i: (0, 0)),
            pl.BlockSpec((3 * D, D), lambda i: (0, 0)),
            pl.BlockSpec((1, D), lambda i: (0, 0)),
        ],
        out_specs=[
            pl.BlockSpec((blk, ROW), lambda i: (i, 0)),
            pl.BlockSpec((blk, ROW), lambda i: (i, 0)),
        ],
        out_shape=[
            jax.ShapeDtypeStruct((N_NODES, ROW), jnp.float32),
            jax.ShapeDtypeStruct((N_NODES, ROW), jnp.float32),
        ],
    )(h_src, h_dst, rel2d, W1, b1_2d)


def _hsum(v):
    s = [v[i] for i in range(NL)]
    while len(s) > 1:
        s = [s[i] + s[i + 1] for i in range(0, len(s) - 1, 2)] + (
            [s[-1]] if len(s) % 2 else [])
    return s[0]


def _edge_body(src_idx_hbm, dst_idx_hbm, stab_hbm, dtab_hbm, w2_hbm, b2_hbm,
               out_hbm, sidx, didx, srows, drows, w2v, b2v, obuf, sem):
    wid = lax.axis_index("s") * NC + lax.axis_index("c")
    base = wid * EPW

    pltpu.sync_copy(w2_hbm, w2v)
    pltpu.sync_copy(b2_hbm, b2v)
    w2r = [w2v[pl.ds(NL * k, NL)] for k in range(D // NL)]
    b2r = b2v[...]
    iota = lax.iota(jnp.int32, NL)
    zero = jnp.zeros((NL,), jnp.float32)

    def step(s, _):
        eb = base + s * K
        pltpu.sync_copy(src_idx_hbm.at[pl.ds(eb, K)], sidx)
        pltpu.sync_copy(dst_idx_hbm.at[pl.ds(eb, K)], didx)
        pltpu.async_copy(stab_hbm.at[sidx], srows, sem).wait()
        pltpu.async_copy(dtab_hbm.at[didx], drows, sem).wait()

        def group(g, _):
            def edge(j, carry):
                gvec, pvec = carry
                jj = g * NL + j
                acc_g = zero
                acc_p = zero
                for k in range(D // NL):
                    a = srows[jj, pl.ds(NL * k, NL)]
                    b = drows[jj, pl.ds(NL * k, NL)]
                    sv = srows[jj, pl.ds(D + NL * k, NL)]
                    hd = drows[jj, pl.ds(D + NL * k, NL)]
                    h = a + b
                    lh = jnp.maximum(h, 0.2 * h)
                    acc_g = acc_g + lh * w2r[k]
                    acc_p = acc_p + sv * hd
                m = iota == j
                gvec = jnp.where(m, _hsum(acc_g), gvec)
                pvec = jnp.where(m, _hsum(acc_p), pvec)
                return gvec, pvec

            gvec, pvec = lax.fori_loop(0, NL, edge, (zero, zero))
            gate = 1.0 / (1.0 + jnp.exp(-(gvec + b2r)))
            obuf[pl.ds(g * NL, NL)] = gate * pvec
            return 0

        lax.fori_loop(0, K // NL, group, 0)
        pltpu.sync_copy(obuf, out_hbm.at[pl.ds(eb, K)])
        return 0

    lax.fori_loop(0, STEPS, step, 0)


def _edge_kernel(src_idx, dst_idx, stab, dtab, w2, b2vec):
    mesh = plsc.VectorSubcoreMesh(core_axis_name="c", subcore_axis_name="s")
    return pl.kernel(
        _edge_body,
        out_type=jax.ShapeDtypeStruct((N_EDGES,), jnp.float32),
        mesh=mesh,
        scratch_types=[
            pltpu.VMEM((K,), jnp.int32),
            pltpu.VMEM((K,), jnp.int32),
            pltpu.VMEM((K, ROW), jnp.float32),
            pltpu.VMEM((K, ROW), jnp.float32),
            pltpu.VMEM((D,), jnp.float32),
            pltpu.VMEM((NL,), jnp.float32),
            pltpu.VMEM((K,), jnp.float32),
            pltpu.SemaphoreType.DMA,
        ],
    )(src_idx, dst_idx, stab, dtab, w2, b2vec)


def kernel(edge_index, h_src, h_dst, rel_weight, W1, b1, W2, b2):
    src_idx = edge_index[0].astype(jnp.int32)
    dst_idx = edge_index[1].astype(jnp.int32)
    rel2d = rel_weight.reshape(1, D)
    b1_2d = b1.reshape(1, D)
    stab, dtab = _build_tables(h_src, h_dst, rel2d, W1, b1_2d)
    w2 = W2.reshape(D)
    b2vec = jnp.broadcast_to(b2.reshape(()), (NL,))
    return _edge_kernel(src_idx, dst_idx, stab, dtab, w2, b2vec)


# speedup vs baseline: 4.2718x; 1.5501x over previous
import functools

import jax
import jax.numpy as jnp
from jax import lax
from jax.experimental import pallas as pl
from jax.experimental.pallas import tpu as pltpu
from jax.experimental.pallas import tpu_sc as plsc

N_NODES = 10000
N_EDGES = 320000
D = 128
ROW = 2 * D

NC, NS, NL = 2, 16, 16
NW = NC * NS
EPW = N_EDGES // NW
K = 80
STEPS = EPW // K


def _tab_body(hs_ref, hd_ref, rel_ref, w1_ref, b1_ref, stab_ref, dtab_ref):
    w1a = w1_ref[0:D, :]
    w1b = w1_ref[D:2 * D, :]
    w1c = w1_ref[2 * D:3 * D, :]
    rel = rel_ref[:]
    c = jnp.dot(rel, w1c, preferred_element_type=jnp.float32) + b1_ref[:]
    hs = hs_ref[:]
    hd = hd_ref[:]
    stab_ref[:, 0:D] = jnp.dot(hs, w1a, preferred_element_type=jnp.float32) + c
    stab_ref[:, D:ROW] = hs * rel
    dtab_ref[:, 0:D] = jnp.dot(hd, w1b, preferred_element_type=jnp.float32)
    dtab_ref[:, D:ROW] = hd


def _build_tables(h_src, h_dst, rel2d, W1, b1_2d):
    blk = 1000
    grid = (N_NODES // blk,)
    return pl.pallas_call(
        _tab_body,
        grid=grid,
        in_specs=[
            pl.BlockSpec((blk, D), lambda i: (i, 0)),
            pl.BlockSpec((blk, D), lambda i: (i, 0)),
            pl.BlockSpec((1, D), lambda i: (0, 0)),
            pl.BlockSpec((3 * D, D), lambda i: (0, 0)),
            pl.BlockSpec((1, D), lambda i: (0, 0)),
        ],
        out_specs=[
            pl.BlockSpec((blk, ROW), lambda i: (i, 0)),
            pl.BlockSpec((blk, ROW), lambda i: (i, 0)),
        ],
        out_shape=[
            jax.ShapeDtypeStruct((N_NODES, ROW), jnp.float32),
            jax.ShapeDtypeStruct((N_NODES, ROW), jnp.float32),
        ],
    )(h_src, h_dst, rel2d, W1, b1_2d)


UNROLL = 2


def _edge_body(src_idx_hbm, dst_idx_hbm, stab_hbm, dtab_hbm, w2_hbm, b2_hbm,
               out_hbm, sidx, didx, srows, drows, w2v, b2v, fbuf, obuf,
               sem0, sem1):
    wid = lax.axis_index("s") * NC + lax.axis_index("c")
    base = wid * EPW
    sems = (sem0, sem1)

    pltpu.sync_copy(w2_hbm, w2v)
    pltpu.sync_copy(b2_hbm, b2v)
    w2r = [w2v[pl.ds(NL * k, NL)] for k in range(D // NL)]
    b2r = b2v[...]
    iota = lax.iota(jnp.int32, NL)
    zero = jnp.zeros((NL,), jnp.float32)

    pltpu.sync_copy(src_idx_hbm.at[pl.ds(base, EPW)], sidx)
    pltpu.sync_copy(dst_idx_hbm.at[pl.ds(base, EPW)], didx)

    def fetch(s, b):
        pltpu.async_copy(stab_hbm.at[sidx.at[pl.ds(s * K, K)]], srows.at[b],
                         sems[b])
        pltpu.async_copy(dtab_hbm.at[didx.at[pl.ds(s * K, K)]], drows.at[b],
                         sems[b])

    def wait_fetch(b):
        pltpu.make_async_copy(stab_hbm.at[sidx.at[pl.ds(0, K)]], srows.at[b],
                              sems[b]).wait()
        pltpu.make_async_copy(dtab_hbm.at[didx.at[pl.ds(0, K)]], drows.at[b],
                              sems[b]).wait()

    def hsum4(acc, fb, off):
        fb[pl.ds(off, NL)] = acc
        t1 = acc + fb[pl.ds(off + 8, NL)]
        fb[pl.ds(off, NL)] = t1
        t2 = t1 + fb[pl.ds(off + 4, NL)]
        return (t2[0] + t2[1]) + (t2[2] + t2[3])

    def edge_work(b, jj, u):
        acc_g = None
        acc_p = None
        for k in range(D // NL):
            a = srows[b, jj, pl.ds(NL * k, NL)]
            bb = drows[b, jj, pl.ds(NL * k, NL)]
            sv = srows[b, jj, pl.ds(D + NL * k, NL)]
            hd = drows[b, jj, pl.ds(D + NL * k, NL)]
            h = a + bb
            lh = jnp.maximum(h, 0.2 * h)
            gterm = lh * w2r[k]
            pterm = sv * hd
            acc_g = gterm if acc_g is None else acc_g + gterm
            acc_p = pterm if acc_p is None else acc_p + pterm
        gs = hsum4(acc_g, fbuf.at[u], 0)
        ps = hsum4(acc_p, fbuf.at[u], 32)
        return gs, ps

    def compute(s, b):
        eb = base + s * K
        for g in range(K // NL):
            def edge2(j2, carry):
                gvec, pvec = carry
                for u in range(UNROLL):
                    j = j2 * UNROLL + u
                    gs, ps = edge_work(b, g * NL + j, u)
                    m = iota == j
                    gvec = jnp.where(m, gs, gvec)
                    pvec = jnp.where(m, ps, pvec)
                return gvec, pvec

            gvec, pvec = lax.fori_loop(0, NL // UNROLL, edge2, (zero, zero))
            gate = 1.0 / (1.0 + jnp.exp(-(gvec + b2r)))
            obuf[pl.ds(g * NL, NL)] = gate * pvec
        pltpu.sync_copy(obuf, out_hbm.at[pl.ds(eb, K)])

    fetch(0, 0)

    def pair(s2, _):
        for b in range(2):
            s = 2 * s2 + b
            fetch(s + 1, 1 - b)
            wait_fetch(b)
            compute(s, b)
        return 0

    lax.fori_loop(0, (STEPS - 1) // 2, pair, 0)
    wait_fetch(0)
    compute(STEPS - 1, 0)


def _edge_kernel(src_idx, dst_idx, stab, dtab, w2, b2vec):
    mesh = plsc.VectorSubcoreMesh(core_axis_name="c", subcore_axis_name="s")
    return pl.kernel(
        _edge_body,
        out_type=jax.ShapeDtypeStruct((N_EDGES,), jnp.float32),
        mesh=mesh,
        scratch_types=[
            pltpu.VMEM((EPW,), jnp.int32),
            pltpu.VMEM((EPW,), jnp.int32),
            pltpu.VMEM((2, K, ROW), jnp.float32),
            pltpu.VMEM((2, K, ROW), jnp.float32),
            pltpu.VMEM((D,), jnp.float32),
            pltpu.VMEM((NL,), jnp.float32),
            pltpu.VMEM((UNROLL, 64), jnp.float32),
            pltpu.VMEM((K,), jnp.float32),
            pltpu.SemaphoreType.DMA,
            pltpu.SemaphoreType.DMA,
        ],
    )(src_idx, dst_idx, stab, dtab, w2, b2vec)


def kernel(edge_index, h_src, h_dst, rel_weight, W1, b1, W2, b2):
    src_idx = edge_index[0].astype(jnp.int32)
    dst_idx = edge_index[1].astype(jnp.int32)
    rel2d = rel_weight.reshape(1, D)
    b1_2d = b1.reshape(1, D)
    stab, dtab = _build_tables(h_src, h_dst, rel2d, W1, b1_2d)
    w2 = W2.reshape(D)
    b2vec = jnp.broadcast_to(b2.reshape(()), (NL,))
    return _edge_kernel(src_idx, dst_idx, stab, dtab, w2, b2vec)


# speedup vs baseline: 4.3462x; 1.0174x over previous
import functools

import jax
import jax.numpy as jnp
from jax import lax
from jax.experimental import pallas as pl
from jax.experimental.pallas import tpu as pltpu
from jax.experimental.pallas import tpu_sc as plsc

N_NODES = 10000
N_EDGES = 320000
D = 128
ROW = 2 * D

NC, NS, NL = 2, 16, 16
NW = NC * NS
EPW = N_EDGES // NW
K = 80
STEPS = EPW // K


def _tab_body(hs_ref, hd_ref, rel_ref, w1_ref, b1_ref, stab_ref, dtab_ref):
    w1a = w1_ref[0:D, :]
    w1b = w1_ref[D:2 * D, :]
    w1c = w1_ref[2 * D:3 * D, :]
    rel = rel_ref[:]
    c = jnp.dot(rel, w1c, preferred_element_type=jnp.float32) + b1_ref[:]
    hs = hs_ref[:]
    hd = hd_ref[:]
    stab_ref[:, 0:D] = jnp.dot(hs, w1a, preferred_element_type=jnp.float32) + c
    stab_ref[:, D:ROW] = hs * rel
    dtab_ref[:, 0:D] = jnp.dot(hd, w1b, preferred_element_type=jnp.float32)
    dtab_ref[:, D:ROW] = hd


def _build_tables(h_src, h_dst, rel2d, W1, b1_2d):
    blk = 1000
    grid = (N_NODES // blk,)
    return pl.pallas_call(
        _tab_body,
        grid=grid,
        in_specs=[
            pl.BlockSpec((blk, D), lambda i: (i, 0)),
            pl.BlockSpec((blk, D), lambda i: (i, 0)),
            pl.BlockSpec((1, D), lambda i: (0, 0)),
            pl.BlockSpec((3 * D, D), lambda i: (0, 0)),
            pl.BlockSpec((1, D), lambda i: (0, 0)),
        ],
        out_specs=[
            pl.BlockSpec((blk, ROW), lambda i: (i, 0)),
            pl.BlockSpec((blk, ROW), lambda i: (i, 0)),
        ],
        out_shape=[
            jax.ShapeDtypeStruct((N_NODES, ROW), jnp.float32),
            jax.ShapeDtypeStruct((N_NODES, ROW), jnp.float32),
        ],
    )(h_src, h_dst, rel2d, W1, b1_2d)


UNROLL = 4


def _edge_body(src_idx_hbm, dst_idx_hbm, stab_hbm, dtab_hbm, w2_hbm, b2_hbm,
               out_hbm, sidx, didx, srows, drows, w2v, b2v, fbuf, obuf,
               sem0, sem1):
    wid = lax.axis_index("s") * NC + lax.axis_index("c")
    base = wid * EPW
    sems = (sem0, sem1)

    pltpu.sync_copy(w2_hbm, w2v)
    pltpu.sync_copy(b2_hbm, b2v)
    w2r = [w2v[pl.ds(NL * k, NL)] for k in range(D // NL)]
    b2r = b2v[...]
    iota = lax.iota(jnp.int32, NL)
    zero = jnp.zeros((NL,), jnp.float32)

    pltpu.sync_copy(src_idx_hbm.at[pl.ds(base, EPW)], sidx)
    pltpu.sync_copy(dst_idx_hbm.at[pl.ds(base, EPW)], didx)

    def fetch(s, b):
        pltpu.async_copy(stab_hbm.at[sidx.at[pl.ds(s * K, K)]], srows.at[b],
                         sems[b])
        pltpu.async_copy(dtab_hbm.at[didx.at[pl.ds(s * K, K)]], drows.at[b],
                         sems[b])

    def wait_fetch(b):
        pltpu.make_async_copy(stab_hbm.at[sidx.at[pl.ds(0, K)]], srows.at[b],
                              sems[b]).wait()
        pltpu.make_async_copy(dtab_hbm.at[didx.at[pl.ds(0, K)]], drows.at[b],
                              sems[b]).wait()

    def hsum4(acc, fb, off):
        fb[pl.ds(off, NL)] = acc
        t1 = acc + fb[pl.ds(off + 8, NL)]
        fb[pl.ds(off, NL)] = t1
        t2 = t1 + fb[pl.ds(off + 4, NL)]
        return (t2[0] + t2[1]) + (t2[2] + t2[3])

    def edge_work(b, jj, u):
        acc_g = None
        acc_p = None
        for k in range(D // NL):
            a = srows[b, jj, pl.ds(NL * k, NL)]
            bb = drows[b, jj, pl.ds(NL * k, NL)]
            sv = srows[b, jj, pl.ds(D + NL * k, NL)]
            hd = drows[b, jj, pl.ds(D + NL * k, NL)]
            h = a + bb
            lh = jnp.maximum(h, 0.2 * h)
            gterm = lh * w2r[k]
            pterm = sv * hd
            acc_g = gterm if acc_g is None else acc_g + gterm
            acc_p = pterm if acc_p is None else acc_p + pterm
        gs = hsum4(acc_g, fbuf.at[u], 0)
        ps = hsum4(acc_p, fbuf.at[u], 32)
        return gs, ps

    def compute(s, b):
        eb = base + s * K
        for g in range(K // NL):
            def edge2(j2, carry):
                gvec, pvec = carry
                for u in range(UNROLL):
                    j = j2 * UNROLL + u
                    gs, ps = edge_work(b, g * NL + j, u)
                    m = iota == j
                    gvec = jnp.where(m, gs, gvec)
                    pvec = jnp.where(m, ps, pvec)
                return gvec, pvec

            gvec, pvec = lax.fori_loop(0, NL // UNROLL, edge2, (zero, zero))
            gate = 1.0 / (1.0 + jnp.exp(-(gvec + b2r)))
            obuf[pl.ds(g * NL, NL)] = gate * pvec
        pltpu.sync_copy(obuf, out_hbm.at[pl.ds(eb, K)])

    fetch(0, 0)

    def pair(s2, _):
        for b in range(2):
            s = 2 * s2 + b
            fetch(s + 1, 1 - b)
            wait_fetch(b)
            compute(s, b)
        return 0

    lax.fori_loop(0, (STEPS - 1) // 2, pair, 0)
    wait_fetch(0)
    compute(STEPS - 1, 0)


def _edge_kernel(src_idx, dst_idx, stab, dtab, w2, b2vec):
    mesh = plsc.VectorSubcoreMesh(core_axis_name="c", subcore_axis_name="s")
    return pl.kernel(
        _edge_body,
        out_type=jax.ShapeDtypeStruct((N_EDGES,), jnp.float32),
        mesh=mesh,
        scratch_types=[
            pltpu.VMEM((EPW,), jnp.int32),
            pltpu.VMEM((EPW,), jnp.int32),
            pltpu.VMEM((2, K, ROW), jnp.float32),
            pltpu.VMEM((2, K, ROW), jnp.float32),
            pltpu.VMEM((D,), jnp.float32),
            pltpu.VMEM((NL,), jnp.float32),
            pltpu.VMEM((UNROLL, 64), jnp.float32),
            pltpu.VMEM((K,), jnp.float32),
            pltpu.SemaphoreType.DMA,
            pltpu.SemaphoreType.DMA,
        ],
    )(src_idx, dst_idx, stab, dtab, w2, b2vec)


def kernel(edge_index, h_src, h_dst, rel_weight, W1, b1, W2, b2):
    src_idx = edge_index[0].astype(jnp.int32)
    dst_idx = edge_index[1].astype(jnp.int32)
    rel2d = rel_weight.reshape(1, D)
    b1_2d = b1.reshape(1, D)
    stab, dtab = _build_tables(h_src, h_dst, rel2d, W1, b1_2d)
    w2 = W2.reshape(D)
    b2vec = jnp.broadcast_to(b2.reshape(()), (NL,))
    return _edge_kernel(src_idx, dst_idx, stab, dtab, w2, b2vec)


# speedup vs baseline: 5.4855x; 1.2621x over previous
import functools

import jax
import jax.numpy as jnp
from jax import lax
from jax.experimental import pallas as pl
from jax.experimental.pallas import tpu as pltpu
from jax.experimental.pallas import tpu_sc as plsc

N_NODES = 10000
N_EDGES = 320000
D = 128
ROW = 2 * D

NC, NS, NL = 2, 16, 16
NW = NC * NS
EPW = N_EDGES // NW
K = 80
STEPS = EPW // K


def _tab_body(hs_ref, hd_ref, rel_ref, w1_ref, b1_ref, stab_ref, dtab_ref):
    w1a = w1_ref[0:D, :]
    w1b = w1_ref[D:2 * D, :]
    w1c = w1_ref[2 * D:3 * D, :]
    rel = rel_ref[:]
    c = jnp.dot(rel, w1c, preferred_element_type=jnp.float32) + b1_ref[:]
    hs = hs_ref[:]
    hd = hd_ref[:]
    stab_ref[:, 0:D] = jnp.dot(hs, w1a, preferred_element_type=jnp.float32) + c
    stab_ref[:, D:ROW] = hs * rel
    dtab_ref[:, 0:D] = jnp.dot(hd, w1b, preferred_element_type=jnp.float32)
    dtab_ref[:, D:ROW] = hd


def _build_tables(h_src, h_dst, rel2d, W1, b1_2d):
    blk = 1000
    grid = (N_NODES // blk,)
    return pl.pallas_call(
        _tab_body,
        grid=grid,
        in_specs=[
            pl.BlockSpec((blk, D), lambda i: (i, 0)),
            pl.BlockSpec((blk, D), lambda i: (i, 0)),
            pl.BlockSpec((1, D), lambda i: (0, 0)),
            pl.BlockSpec((3 * D, D), lambda i: (0, 0)),
            pl.BlockSpec((1, D), lambda i: (0, 0)),
        ],
        out_specs=[
            pl.BlockSpec((blk, ROW), lambda i: (i, 0)),
            pl.BlockSpec((blk, ROW), lambda i: (i, 0)),
        ],
        out_shape=[
            jax.ShapeDtypeStruct((N_NODES, ROW), jnp.float32),
            jax.ShapeDtypeStruct((N_NODES, ROW), jnp.float32),
        ],
    )(h_src, h_dst, rel2d, W1, b1_2d)


UNROLL = 2


def _edge_body(src_idx_hbm, dst_idx_hbm, stab_hbm, dtab_hbm, w2_hbm, b2_hbm,
               out_hbm, sidx, didx, srows, drows, w2v, b2v, obuf,
               sem0, sem1):
    wid = lax.axis_index("s") * NC + lax.axis_index("c")
    base = wid * EPW
    sems = (sem0, sem1)

    pltpu.sync_copy(w2_hbm, w2v)
    pltpu.sync_copy(b2_hbm, b2v)
    w2r = [w2v[pl.ds(NL * k, NL)] for k in range(D // NL)]
    b2r = b2v[...]
    iota = lax.iota(jnp.int32, NL)
    zero = jnp.zeros((NL,), jnp.float32)

    pltpu.sync_copy(src_idx_hbm.at[pl.ds(base, EPW)], sidx)
    pltpu.sync_copy(dst_idx_hbm.at[pl.ds(base, EPW)], didx)

    def fetch(s, b):
        pltpu.async_copy(stab_hbm.at[sidx.at[pl.ds(s * K, K)]], srows.at[b],
                         sems[b])
        pltpu.async_copy(dtab_hbm.at[didx.at[pl.ds(s * K, K)]], drows.at[b],
                         sems[b])

    def wait_fetch(b):
        pltpu.make_async_copy(stab_hbm.at[sidx.at[pl.ds(0, K)]], srows.at[b],
                              sems[b]).wait()
        pltpu.make_async_copy(dtab_hbm.at[didx.at[pl.ds(0, K)]], drows.at[b],
                              sems[b]).wait()

    def hsum_bcast(v):
        for k in (8, 4, 2, 1):
            v = v + v.at[iota ^ k].get(mode="promise_in_bounds")
        return v

    def edge_work(b, jj, u):
        acc_g = None
        acc_p = None
        for k in range(D // NL):
            a = srows[b, jj, pl.ds(NL * k, NL)]
            bb = drows[b, jj, pl.ds(NL * k, NL)]
            sv = srows[b, jj, pl.ds(D + NL * k, NL)]
            hd = drows[b, jj, pl.ds(D + NL * k, NL)]
            h = a + bb
            lh = jnp.maximum(h, 0.2 * h)
            gterm = lh * w2r[k]
            pterm = sv * hd
            acc_g = gterm if acc_g is None else acc_g + gterm
            acc_p = pterm if acc_p is None else acc_p + pterm
        return hsum_bcast(acc_g), hsum_bcast(acc_p)

    def compute(s, b):
        eb = base + s * K
        for g in range(K // NL):
            def edge2(j2, carry):
                gvec, pvec = carry
                for u in range(UNROLL):
                    j = j2 * UNROLL + u
                    gs, ps = edge_work(b, g * NL + j, u)
                    m = iota == j
                    gvec = jnp.where(m, gs, gvec)
                    pvec = jnp.where(m, ps, pvec)
                return gvec, pvec

            gvec, pvec = lax.fori_loop(0, NL // UNROLL, edge2, (zero, zero))
            gate = 1.0 / (1.0 + jnp.exp(-(gvec + b2r)))
            obuf[pl.ds(g * NL, NL)] = gate * pvec
        pltpu.sync_copy(obuf, out_hbm.at[pl.ds(eb, K)])

    fetch(0, 0)

    def pair(s2, _):
        for b in range(2):
            s = 2 * s2 + b
            fetch(s + 1, 1 - b)
            wait_fetch(b)
            compute(s, b)
        return 0

    lax.fori_loop(0, (STEPS - 1) // 2, pair, 0)
    wait_fetch(0)
    compute(STEPS - 1, 0)


def _edge_kernel(src_idx, dst_idx, stab, dtab, w2, b2vec):
    mesh = plsc.VectorSubcoreMesh(core_axis_name="c", subcore_axis_name="s")
    return pl.kernel(
        _edge_body,
        out_type=jax.ShapeDtypeStruct((N_EDGES,), jnp.float32),
        mesh=mesh,
        scratch_types=[
            pltpu.VMEM((EPW,), jnp.int32),
            pltpu.VMEM((EPW,), jnp.int32),
            pltpu.VMEM((2, K, ROW), jnp.float32),
            pltpu.VMEM((2, K, ROW), jnp.float32),
            pltpu.VMEM((D,), jnp.float32),
            pltpu.VMEM((NL,), jnp.float32),
            pltpu.VMEM((K,), jnp.float32),
            pltpu.SemaphoreType.DMA,
            pltpu.SemaphoreType.DMA,
        ],
    )(src_idx, dst_idx, stab, dtab, w2, b2vec)


def kernel(edge_index, h_src, h_dst, rel_weight, W1, b1, W2, b2):
    src_idx = edge_index[0].astype(jnp.int32)
    dst_idx = edge_index[1].astype(jnp.int32)
    rel2d = rel_weight.reshape(1, D)
    b1_2d = b1.reshape(1, D)
    stab, dtab = _build_tables(h_src, h_dst, rel2d, W1, b1_2d)
    w2 = W2.reshape(D)
    b2vec = jnp.broadcast_to(b2.reshape(()), (NL,))
    return _edge_kernel(src_idx, dst_idx, stab, dtab, w2, b2vec)
